# baseline (speedup 1.0000x reference)
import jax
import jax.numpy as jnp
from jax.experimental import pallas as pl
from jax.experimental.pallas import tpu as pltpu

_B, _L, _C = 4096, 20, 1000
_N = _B * _L
_ROWS = 1024
_NSTEP = _N // _ROWS
_NBUF = 8
_SPS = _NSTEP // _NBUF


def _seed(o_ref):
    pass


def _onehot_multibuf(x_ref, seed_ref, o_ref, *scratch):
    bufs = scratch[:_NBUF]
    sems = scratch[_NBUF:]
    col = jax.lax.broadcasted_iota(jnp.int32, (_ROWS, _C + 1), 1)
    dsts = [o_ref if b % 2 == 0 else seed_ref for b in range(_NBUF)]

    def step(go, carry):
        for b in range(_NBUF):
            i = b * _SPS + go
            buf, sem, dst = bufs[b], sems[b], dsts[b]

            @pl.when(go >= 1)
            def _wait_prev():
                pltpu.make_async_copy(
                    buf,
                    dst.at[pl.ds((i - 1) * _ROWS, _ROWS), :],
                    sem,
                ).wait()

            xb = x_ref[:, pl.ds(i * _ROWS, _ROWS)]
            xt = jax.lax.transpose(xb, (1, 0))
            act = xt[:, 0:1].astype(jnp.int32)
            dur = xt[:, 1:2]
            buf[...] = (col == act).astype(jnp.float32)
            buf[:, _C:_C + 1] = dur
            pltpu.make_async_copy(
                buf,
                dst.at[pl.ds(i * _ROWS, _ROWS), :],
                sem,
            ).start()
        return carry

    jax.lax.fori_loop(0, _SPS, step, 0)

    for b in range(_NBUF):
        i = (b + 1) * _SPS - 1
        pltpu.make_async_copy(
            bufs[b],
            dsts[b].at[pl.ds(i * _ROWS, _ROWS), :],
            sems[b],
        ).wait()


def kernel(x):
    xt = x.reshape(_N, 2).T
    seed = pl.pallas_call(
        _seed,
        out_specs=pl.BlockSpec(memory_space=pl.ANY),
        out_shape=jax.ShapeDtypeStruct((_N, _C + 1), jnp.float32),
    )()
    out = pl.pallas_call(
        _onehot_multibuf,
        in_specs=[pl.BlockSpec(memory_space=pltpu.VMEM),
                  pl.BlockSpec(memory_space=pl.ANY)],
        out_specs=pl.BlockSpec(memory_space=pl.ANY),
        out_shape=jax.ShapeDtypeStruct((_N, _C + 1), jnp.float32),
        input_output_aliases={1: 0},
        scratch_shapes=(
            [pltpu.VMEM((_ROWS, _C + 1), jnp.float32) for _ in range(_NBUF)]
            + [pltpu.SemaphoreType.DMA for _ in range(_NBUF)]
        ),
    )(xt, seed)
    return out.reshape(_B, _L, _C + 1)
